# baseline (speedup 1.0000x reference)
import functools

import jax
import jax.numpy as jnp
from jax import lax
from jax.experimental import pallas as pl
from jax.experimental.pallas import tpu as pltpu
from jax.experimental.pallas import tpu_sc as plsc

APP_ROWS = 1000000
ENT_ROWS = 1000000
BATCH = 16384
EMBED_DIM = 64
LANES = 16
NUM_CORES = 2
NUM_SUBCORES = 16
NUM_WORKERS = NUM_CORES * NUM_SUBCORES
ROWS_PER_WORKER = BATCH // NUM_WORKERS
CHUNK = 256
NUM_CHUNKS = ROWS_PER_WORKER // CHUNK
NUM_DCHUNKS = EMBED_DIM // LANES
PART_PER_WORKER = ROWS_PER_WORKER * LANES
PART_TOTAL = BATCH * LANES


def _sc_pair_partials(pos_app, pos_entity, neg_app, neg_entity,
                      app_emb, entity_emb):
    mesh = plsc.VectorSubcoreMesh(core_axis_name="c", subcore_axis_name="s")

    @functools.partial(
        pl.kernel,
        out_type=[jax.ShapeDtypeStruct((PART_TOTAL,), jnp.float32),
                  jax.ShapeDtypeStruct((PART_TOTAL,), jnp.float32)],
        mesh=mesh,
        scratch_types=[
            pltpu.VMEM((ROWS_PER_WORKER,), jnp.int32),
            pltpu.VMEM((ROWS_PER_WORKER,), jnp.int32),
            pltpu.VMEM((CHUNK // 8, 8, EMBED_DIM), jnp.float32),
            pltpu.VMEM((CHUNK // 8, 8, EMBED_DIM), jnp.float32),
            pltpu.VMEM((PART_PER_WORKER,), jnp.float32),
            pltpu.SemaphoreType.DMA,
        ],
    )
    def sc_kernel(pa_hbm, pe_hbm, na_hbm, ne_hbm, ta_hbm, te_hbm,
                  out_pos, out_neg, idx_a, idx_b, rows_a, rows_b,
                  part, sem):
        wid = lax.axis_index("s") * NUM_CORES + lax.axis_index("c")
        base = wid * ROWS_PER_WORKER
        tav = ta_hbm.reshape(APP_ROWS // 8, 8, EMBED_DIM)
        tev = te_hbm.reshape(ENT_ROWS // 8, 8, EMBED_DIM)

        def do_pair(ia_hbm, ib_hbm, out_hbm):
            pltpu.sync_copy(ia_hbm.at[pl.ds(base, ROWS_PER_WORKER)], idx_a)
            pltpu.sync_copy(ib_hbm.at[pl.ds(base, ROWS_PER_WORKER)], idx_b)

            for c in range(NUM_CHUNKS):
                cbase = c * CHUNK

                @pl.loop(0, CHUNK // LANES)
                def _(g):
                    iva = idx_a[pl.ds(cbase + g * LANES, LANES)]
                    ivb = idx_b[pl.ds(cbase + g * LANES, LANES)]
                    for k in range(LANES):
                        q = g * 2 + k // 8
                        ia = iva[k]
                        ib = ivb[k]
                        pltpu.async_copy(tav.at[ia >> 3, ia & 7],
                                         rows_a.at[q, k % 8], sem)
                        pltpu.async_copy(tev.at[ib >> 3, ib & 7],
                                         rows_b.at[q, k % 8], sem)

                @pl.loop(0, CHUNK)
                def _(j):
                    pltpu.make_async_copy(tav.at[0, 0], rows_a.at[0, 0],
                                          sem).wait()
                    pltpu.make_async_copy(tev.at[0, 0], rows_b.at[0, 0],
                                          sem).wait()

                @pl.loop(0, CHUNK // 8)
                def _(q):
                    for s in range(8):
                        acc = (rows_a[q, s, pl.ds(0, LANES)] *
                               rows_b[q, s, pl.ds(0, LANES)])
                        for d in range(1, NUM_DCHUNKS):
                            sl = pl.ds(d * LANES, LANES)
                            acc = acc + rows_a[q, s, sl] * rows_b[q, s, sl]
                        part[pl.ds((cbase + q * 8 + s) * LANES, LANES)] = acc

            pltpu.sync_copy(part, out_hbm.at[pl.ds(wid * PART_PER_WORKER,
                                                   PART_PER_WORKER)])

        do_pair(pa_hbm, pe_hbm, out_pos)
        do_pair(na_hbm, ne_hbm, out_neg)

    return sc_kernel(pos_app, pos_entity, neg_app, neg_entity,
                     app_emb, entity_emb)


def _tc_loss(pos_part, neg_part):
    def body(p_ref, n_ref, o_ref):
        lane = lax.broadcasted_iota(jnp.int32, (128, 8), 0)
        grp = lax.broadcasted_iota(jnp.int32, (128, 8), 1)
        sel = (lane // LANES == grp).astype(jnp.float32)
        dn = (((1,), (0,)), ((), ()))
        ps = lax.dot_general(p_ref[...], sel, dn,
                             preferred_element_type=jnp.float32)
        ns = lax.dot_general(n_ref[...], sel, dn,
                             preferred_element_type=jnp.float32)
        pls = jnp.minimum(ps, 0.0) - jnp.log1p(jnp.exp(-jnp.abs(ps)))
        nls = jnp.minimum(-ns, 0.0) - jnp.log1p(jnp.exp(-jnp.abs(ns)))
        o_ref[0, 0] = -(jnp.sum(pls) + jnp.sum(nls))

    out = pl.pallas_call(
        body,
        out_shape=jax.ShapeDtypeStruct((1, 1), jnp.float32),
        out_specs=pl.BlockSpec(memory_space=pltpu.SMEM),
    )(pos_part.reshape(PART_TOTAL // 128, 128),
      neg_part.reshape(PART_TOTAL // 128, 128))
    return out[0, 0]


def kernel(pos_app, pos_entity, neg_app, neg_entity, app_emb, entity_emb):
    iota = lax.iota(jnp.int32, BATCH)
    pos_app, pperm = lax.sort([pos_app.astype(jnp.int32), iota], num_keys=1)
    pos_entity = jnp.take(pos_entity, pperm, axis=0)
    neg_app, nperm = lax.sort([neg_app.astype(jnp.int32), iota], num_keys=1)
    neg_entity = jnp.take(neg_entity, nperm, axis=0)
    pos_part, neg_part = _sc_pair_partials(
        pos_app.astype(jnp.int32), pos_entity.astype(jnp.int32),
        neg_app.astype(jnp.int32), neg_entity.astype(jnp.int32),
        app_emb, entity_emb)
    return _tc_loss(pos_part, neg_part)


# speedup vs baseline: 1.4326x; 1.4326x over previous
import functools

import jax
import jax.numpy as jnp
from jax import lax
from jax.experimental import pallas as pl
from jax.experimental.pallas import tpu as pltpu
from jax.experimental.pallas import tpu_sc as plsc

TABLE_ROWS = 1000000
BATCH = 16384
NLOOK = 2 * BATCH
EMBED_DIM = 64
LANES = 16
NUM_CORES = 2
NUM_SUBCORES = 16
NUM_WORKERS = NUM_CORES * NUM_SUBCORES
EPW = NLOOK // NUM_WORKERS
NGROUPS = EPW // LANES
SLAB = 1024
CUT = (TABLE_ROWS // 128) * 128
MAX_SLAB_BASE = CUT - SLAB
NUM_DCHUNKS = EMBED_DIM // LANES
STAGE_SLOTS = 32


def _sc_extract(sorted_rows, sorted_enc, tab3, tail):
    mesh = plsc.VectorSubcoreMesh(core_axis_name="c", subcore_axis_name="s")

    @functools.partial(
        pl.kernel,
        out_type=jax.ShapeDtypeStruct((NLOOK * EMBED_DIM,), jnp.float32),
        mesh=mesh,
        compiler_params=pltpu.CompilerParams(needs_layout_passes=False),
        scratch_types=[
            pltpu.VMEM((EPW,), jnp.int32),
            pltpu.VMEM((EPW,), jnp.int32),
            pltpu.VMEM((8, 8, SLAB), jnp.float32),
            pltpu.VMEM((64, EMBED_DIM), jnp.float32),
            pltpu.VMEM((STAGE_SLOTS * EMBED_DIM,), jnp.float32),
            pltpu.SMEM((8,), jnp.int32),
            pltpu.SemaphoreType.DMA,
        ],
    )
    def k(rows_hbm, enc_hbm, tab_hbm, tail_hbm, out_hbm,
          rows_v, enc_v, slab, tailbuf, stage, cnt, sem):
        wid = lax.axis_index("s") * NUM_CORES + lax.axis_index("c")
        base = wid * EPW
        pltpu.sync_copy(rows_hbm.at[pl.ds(base, EPW)], rows_v)
        pltpu.sync_copy(enc_hbm.at[pl.ds(base, EPW)], enc_v)
        pltpu.sync_copy(tail_hbm, tailbuf)

        lane16 = lax.iota(jnp.int32, LANES)
        idx_c = [((lane16 + d * LANES) >> 3).astype(jnp.int32)
                 for d in range(NUM_DCHUNKS)]
        idx_s = [((lane16 + d * LANES) & 7).astype(jnp.int32)
                 for d in range(NUM_DCHUNKS)]

        cnt[0] = 0
        cnt[1] = 0

        def extract_entry(r, e, from_tail, lane0):
            fired = cnt[0]
            drained = cnt[1]

            @pl.when(fired - drained >= STAGE_SLOTS)
            def _():
                pltpu.make_async_copy(
                    tab_hbm.at[0, 0, pl.ds(0, EMBED_DIM)],
                    stage.at[pl.ds(0, EMBED_DIM)], sem).wait()
                cnt[1] = drained + 1

            slot = lax.rem(fired, STAGE_SLOTS)
            soff = slot * EMBED_DIM
            for d in range(NUM_DCHUNKS):
                if from_tail:
                    vals = plsc.load_gather(
                        tailbuf, [jnp.full((LANES,), r - CUT, jnp.int32),
                                  lane16 + d * LANES])
                else:
                    vals = plsc.load_gather(
                        slab, [idx_c[d], idx_s[d],
                               jnp.full((LANES,), r - lane0, jnp.int32)])
                stage[pl.ds(soff + d * LANES, LANES)] = vals
            pltpu.async_copy(
                stage.at[pl.ds(soff, EMBED_DIM)],
                out_hbm.at[pl.ds(e * EMBED_DIM, EMBED_DIM)], sem)
            cnt[0] = fired + 1

        def scan_groups(lo, hi, from_tail, lane0):
            @pl.loop(0, NGROUPS)
            def _(g):
                rv = rows_v[pl.ds(g * LANES, LANES)]
                m = jnp.logical_and(rv >= lo, rv < hi).astype(jnp.int32)
                nmatch = plsc.all_reduce_population_count(m != 0)[0]

                @pl.when(nmatch > 0)
                def _():
                    ev = enc_v[pl.ds(g * LANES, LANES)]
                    for kk in range(LANES):
                        mk = m[kk]

                        @pl.when(mk == 1)
                        def _():
                            extract_entry(rv[kk], ev[kk], from_tail, lane0)

        r_first = rows_v[pl.ds(0, LANES)][0]
        r_last = rows_v[pl.ds(EPW - LANES, LANES)][LANES - 1]
        span_base = jnp.minimum((r_first >> 7) << 7, MAX_SLAB_BASE)
        span_end = jnp.minimum(((r_last >> 7) << 7) + 128, CUT)
        n_slabs = jnp.maximum((span_end - span_base + SLAB - 1) // SLAB, 0)

        def slab_body(si, carry):
            lane0 = pl.multiple_of(
                jnp.minimum(span_base + si * SLAB, MAX_SLAB_BASE), 128)
            pltpu.sync_copy(tab_hbm.at[:, :, pl.ds(lane0, SLAB)], slab)
            scan_groups(lane0, lane0 + SLAB, False, lane0)
            return carry

        lax.fori_loop(0, n_slabs, slab_body, 0)

        scan_groups(CUT, TABLE_ROWS, True, 0)

        def drain_body(i, carry):
            pltpu.make_async_copy(
                tab_hbm.at[0, 0, pl.ds(0, EMBED_DIM)],
                stage.at[pl.ds(0, EMBED_DIM)], sem).wait()
            return carry

        lax.fori_loop(cnt[1], cnt[0], drain_body, 0)

    return k(sorted_rows, sorted_enc, tab3, tail)


def _tc_loss(rows_a, rows_b):
    R = NLOOK * EMBED_DIM // 128

    def body(a_ref, b_ref, o_ref):
        prod = a_ref[...] * b_ref[...]
        lane = lax.broadcasted_iota(jnp.int32, (128, 2), 0)
        half = lax.broadcasted_iota(jnp.int32, (128, 2), 1)
        sel = (lane // EMBED_DIM == half).astype(jnp.float32)
        dn = (((1,), (0,)), ((), ()))
        sc = lax.dot_general(prod, sel, dn,
                             preferred_element_type=jnp.float32)
        row = lax.broadcasted_iota(jnp.int32, (R, 2), 0)
        sign = jnp.where(row < R // 2, 1.0, -1.0)
        x = sign * sc
        ls = jnp.minimum(x, 0.0) - jnp.log1p(jnp.exp(-jnp.abs(x)))
        o_ref[0, 0] = -jnp.sum(ls)

    out = pl.pallas_call(
        body,
        out_shape=jax.ShapeDtypeStruct((1, 1), jnp.float32),
        out_specs=pl.BlockSpec(memory_space=pltpu.SMEM),
    )(rows_a.reshape(R, 128), rows_b.reshape(R, 128))
    return out[0, 0]


def kernel(pos_app, pos_entity, neg_app, neg_entity, app_emb, entity_emb):
    iota2 = lax.iota(jnp.int32, NLOOK)
    ia = jnp.concatenate([pos_app.astype(jnp.int32),
                          neg_app.astype(jnp.int32)])
    ib = jnp.concatenate([pos_entity.astype(jnp.int32),
                          neg_entity.astype(jnp.int32)])
    sa, ea = lax.sort([ia, iota2], num_keys=1)
    sb, eb = lax.sort([ib, iota2], num_keys=1)

    a3 = app_emb.T.reshape(8, 8, TABLE_ROWS)
    b3 = entity_emb.T.reshape(8, 8, TABLE_ROWS)
    tail_a = lax.slice(app_emb, (CUT, 0), (TABLE_ROWS, EMBED_DIM))
    tail_b = lax.slice(entity_emb, (CUT, 0), (TABLE_ROWS, EMBED_DIM))

    rows_a = _sc_extract(sa, ea, a3, tail_a)
    rows_b = _sc_extract(sb, eb, b3, tail_b)
    return _tc_loss(rows_a, rows_b)
